# baseline (speedup 1.0000x reference)
import functools

import jax
import jax.numpy as jnp
from jax.experimental import pallas as pl


def _gate_block(e_ref, o_ref):
    e = e_ref[...]
    fine = (e > 0.5).astype(jnp.float32)
    k = jax.lax.broadcasted_iota(jnp.int32, (128, 256), 0)
    j = jax.lax.broadcasted_iota(jnp.int32, (128, 256), 1)
    odd = (j & 1).astype(jnp.float32)
    sel = jnp.where((j // 2) == k, 2.0 * odd - 1.0, 0.0)
    base = 1.0 - odd
    g = jax.lax.dot(fine, sel, preferred_element_type=jnp.float32) + base[:1, :]
    o_ref[...] = g.astype(jnp.int32)


@functools.partial(jax.jit, static_argnames=())
def kernel(entropy):
    flat = entropy.reshape(2048, 128)
    rows_per_block = 256
    grid = 2048 // rows_per_block
    out = pl.pallas_call(
        _gate_block,
        grid=(grid,),
        in_specs=[pl.BlockSpec((rows_per_block, 128), lambda i: (i, 0))],
        out_specs=pl.BlockSpec((rows_per_block, 256), lambda i: (i, 0)),
        out_shape=jax.ShapeDtypeStruct((2048, 256), jnp.int32),
    )(flat)
    return out.reshape(256, 32, 32, 2)


# speedup vs baseline: 36.0179x; 36.0179x over previous
import jax
import jax.numpy as jnp
from jax.experimental import pallas as pl


def _gate_block(e_ref, o_ref):
    e = e_ref[...]
    fine = (e > 0.5).astype(jnp.int32)
    p = jax.lax.broadcasted_iota(jnp.int32, (e.shape[0], 2, 256), 1)
    o_ref[...] = (p == fine[:, None, :]).astype(jnp.int32)


def kernel(entropy):
    et = jnp.transpose(entropy, (1, 2, 0)).reshape(1024, 256)
    blk = 128
    out = pl.pallas_call(
        _gate_block,
        grid=(1024 // blk,),
        in_specs=[pl.BlockSpec((blk, 256), lambda i: (i, 0))],
        out_specs=pl.BlockSpec((blk, 2, 256), lambda i: (i, 0, 0)),
        out_shape=jax.ShapeDtypeStruct((1024, 2, 256), jnp.int32),
    )(et)
    return jnp.transpose(out.reshape(32, 32, 2, 256), (3, 0, 1, 2))


# speedup vs baseline: 38.8733x; 1.0793x over previous
import jax
import jax.numpy as jnp
from jax.experimental import pallas as pl


def _gate_block(e_ref, o_ref):
    e = e_ref[...]
    fine = (e > 0.5).astype(jnp.int32)
    o_ref[:, 0, :] = fine ^ 1
    o_ref[:, 1, :] = fine


def kernel(entropy):
    et = jnp.transpose(entropy, (1, 2, 0)).reshape(1024, 256)
    blk = 128
    out = pl.pallas_call(
        _gate_block,
        grid=(1024 // blk,),
        in_specs=[pl.BlockSpec((blk, 256), lambda i: (i, 0))],
        out_specs=pl.BlockSpec((blk, 2, 256), lambda i: (i, 0, 0)),
        out_shape=jax.ShapeDtypeStruct((1024, 2, 256), jnp.int32),
    )(et)
    return jnp.transpose(out.reshape(32, 32, 2, 256), (3, 0, 1, 2))


# speedup vs baseline: 38.9837x; 1.0028x over previous
import jax
import jax.numpy as jnp
from jax.experimental import pallas as pl
from jax.experimental.pallas import tpu as pltpu


def _gate_block(e_ref, o_ref):
    e = e_ref[...]
    fine = (e > 0.5).astype(jnp.int32)
    o_ref[:, 0, :] = fine ^ 1
    o_ref[:, 1, :] = fine


def kernel(entropy):
    et = jnp.transpose(entropy, (1, 2, 0)).reshape(1024, 256)
    blk = 128
    out = pl.pallas_call(
        _gate_block,
        grid=(1024 // blk,),
        in_specs=[pl.BlockSpec((blk, 256), lambda i: (i, 0))],
        out_specs=pl.BlockSpec((blk, 2, 256), lambda i: (i, 0, 0)),
        out_shape=jax.ShapeDtypeStruct((1024, 2, 256), jnp.int32),
        compiler_params=pltpu.CompilerParams(
            dimension_semantics=("parallel",),
        ),
    )(et)
    return jnp.transpose(out.reshape(32, 32, 2, 256), (3, 0, 1, 2))


# speedup vs baseline: 79.2645x; 2.0333x over previous
import jax
import jax.numpy as jnp
from jax.experimental import pallas as pl
from jax.experimental.pallas import tpu as pltpu


def _gate_block(e_ref, o_ref):
    e = e_ref[...]
    fine = (e > 0.5).astype(jnp.int32)
    o_ref[:, 0, :] = fine ^ 1
    o_ref[:, 1, :] = fine


def kernel(entropy):
    et = jnp.transpose(entropy, (1, 2, 0)).reshape(1024, 256)
    blk = 512
    out = pl.pallas_call(
        _gate_block,
        grid=(1024 // blk,),
        in_specs=[pl.BlockSpec((blk, 256), lambda i: (i, 0))],
        out_specs=pl.BlockSpec((blk, 2, 256), lambda i: (i, 0, 0)),
        out_shape=jax.ShapeDtypeStruct((1024, 2, 256), jnp.int32),
        compiler_params=pltpu.CompilerParams(
            dimension_semantics=("parallel",),
        ),
    )(et)
    return jnp.transpose(out.reshape(32, 32, 2, 256), (3, 0, 1, 2))
